# baseline (speedup 1.0000x reference)
import functools

import jax
import jax.numpy as jnp
import numpy as np
from jax.experimental import pallas as pl
from jax.experimental.pallas import tpu as pltpu

_RADIUS = 60
_R = _RADIUS - 1
_OFFSETS = [(-1, -1), (-1, 0), (0, -1), (0, 0), (0, 1), (1, 0), (1, 1)]
_G = 2 * _R + 3
_ND = _G * _G
_SHIFTS = [_G * di + dk for (di, dk) in _OFFSETS]

_C = 256
_NCHUNK = -(-_ND // _C)
_NDPAD = _NCHUNK * _C


def _build_dense_map():
    rows = []
    s = 0
    for i in range(-_R, _R + 1):
        kmin = max(-_R, i - _R)
        ln = min(_R, i + _R) - kmin + 1
        rows.append((i, kmin, ln, s))
        s += ln
    n_total = s
    dense_idx = np.zeros(n_total, np.int32)
    for (i, kmin, ln, st) in rows:
        d0 = (i + _R + 1) * _G + (kmin + _R + 1)
        dense_idx[st:st + ln] = d0 + np.arange(ln, dtype=np.int32)
    return rows, n_total, dense_idx


_ROWS, _N, _DENSE_IDX = _build_dense_map()


def _conv_body(xp_ref, xc_ref, xn_ref, w_ref, b_ref, o_ref):
    xw = jnp.concatenate([xp_ref[0], xc_ref[0], xn_ref[0]], axis=0)
    acc = jnp.broadcast_to(b_ref[0], (_C, o_ref.shape[2])).astype(jnp.float32)
    for t, s in enumerate(_SHIFTS):
        xs = xw[_C + s:_C + s + _C, :]
        acc = acc + jnp.dot(xs, w_ref[t], preferred_element_type=jnp.float32)
    o_ref[0] = acc


@functools.partial(jax.jit, static_argnums=())
def _hexconv_dense(xd, kernel_weights, bias2d):
    batch = xd.shape[0]
    feat = xd.shape[2]
    out_dim = kernel_weights.shape[2]
    grid = (batch, _NCHUNK)
    ib = pl.BlockSpec((1, _C, feat), lambda b, j: (b, jnp.maximum(j - 1, 0), 0))
    ic = pl.BlockSpec((1, _C, feat), lambda b, j: (b, j, 0))
    inx = pl.BlockSpec((1, _C, feat),
                       lambda b, j: (b, jnp.minimum(j + 1, _NCHUNK - 1), 0))
    wspec = pl.BlockSpec(kernel_weights.shape, lambda b, j: (0, 0, 0))
    bspec = pl.BlockSpec((1, out_dim), lambda b, j: (0, 0))
    ospec = pl.BlockSpec((1, _C, out_dim), lambda b, j: (b, j, 0))
    return pl.pallas_call(
        _conv_body,
        grid=grid,
        in_specs=[ib, ic, inx, wspec, bspec],
        out_specs=ospec,
        out_shape=jax.ShapeDtypeStruct((batch, _NDPAD, out_dim), jnp.float32),
        compiler_params=pltpu.CompilerParams(
            dimension_semantics=("parallel", "arbitrary")),
    )(xd, xd, xd, kernel_weights, bias2d)


def kernel(inputs, kernel_weights, bias):
    dense_idx = jnp.asarray(_DENSE_IDX)
    xd = jnp.zeros((inputs.shape[0], _NDPAD, inputs.shape[2]),
                   jnp.float32).at[:, dense_idx, :].set(inputs)
    yd = _hexconv_dense(xd, kernel_weights, bias.reshape(1, -1))
    return jnp.take(yd, dense_idx, axis=1)
